# Initial kernel scaffold; baseline (speedup 1.0000x reference)
import jax
import jax.numpy as jnp
from jax.experimental import pallas as pl


def kernel(pred, target):
    raise NotImplementedError("write your pallas kernel here")


# speedup vs baseline: 2.0816x; 2.0816x over previous
import math

import jax
import jax.numpy as jnp
from jax import lax
from jax.experimental import pallas as pl
from jax.experimental.pallas import tpu as pltpu

_C = 100000
_B = 1024
_SMOOTH = 0.1
_CONF = 1.0 - _SMOOTH
_EPS = _SMOOTH / (_C - 1)
_K0 = _SMOOTH * math.log(_EPS) + _CONF * math.log(_CONF)

_CB = 2048
_NBLK = (_C + _CB - 1) // _CB


def _body(pred_ref, tgt_ref, out_ref, sumexp_acc, sum_acc, pt_acc):
    j = pl.program_id(0)
    x = pred_ref[...]
    cols = j * _CB + lax.broadcasted_iota(jnp.int32, (_B, _CB), 1)
    valid = cols < _C
    e = jnp.where(valid, jnp.exp(x), 0.0)
    xs = jnp.where(valid, x, 0.0)
    pt = jnp.where(cols == tgt_ref[...], x, 0.0)

    @pl.when(j == 0)
    def _init():
        sumexp_acc[...] = jnp.zeros_like(sumexp_acc)
        sum_acc[...] = jnp.zeros_like(sum_acc)
        pt_acc[...] = jnp.zeros_like(pt_acc)

    sumexp_acc[...] += e.sum(axis=1, keepdims=True)
    sum_acc[...] += xs.sum(axis=1, keepdims=True)
    pt_acc[...] += pt.sum(axis=1, keepdims=True)

    @pl.when(j == _NBLK - 1)
    def _fin():
        lse = jnp.log(sumexp_acc[...])
        total = (
            jnp.sum(lse)
            - _EPS * jnp.sum(sum_acc[...])
            - (_CONF - _EPS) * jnp.sum(pt_acc[...])
        ) / _B + _K0
        out_ref[...] = jnp.reshape(total, (1, 1))


def kernel(pred, target):
    tgt = target.astype(jnp.int32).reshape(_B, 1)
    out = pl.pallas_call(
        _body,
        grid=(_NBLK,),
        in_specs=[
            pl.BlockSpec((_B, _CB), lambda j: (0, j)),
            pl.BlockSpec((_B, 1), lambda j: (0, 0)),
        ],
        out_specs=pl.BlockSpec((1, 1), lambda j: (0, 0)),
        out_shape=jax.ShapeDtypeStruct((1, 1), jnp.float32),
        scratch_shapes=[
            pltpu.VMEM((_B, 1), jnp.float32),
            pltpu.VMEM((_B, 1), jnp.float32),
            pltpu.VMEM((_B, 1), jnp.float32),
        ],
    )(pred, tgt)
    return out[0, 0]
